# baseline (speedup 1.0000x reference)
import jax
import jax.numpy as jnp
from jax import lax
from jax.experimental import pallas as pl
from jax.experimental.pallas import tpu as pltpu
from jax.experimental.pallas import tpu_sc as plsc

D_MODEL = 1024
NC = 2
NS = 16
NW = NC * NS

K = 32


NBUF = 3


def _gather_body(x_hbm, pe_hbm, out_hbm, idx_v, rows_a, rows_b, rows_c,
                 gsem_a, gsem_b, gsem_c, wsem_a, wsem_b, wsem_c):
    c = lax.axis_index("c")
    s = lax.axis_index("s")
    wid = s * NC + c
    n_chunks = idx_v.shape[0]
    n_per_w = n_chunks * idx_v.shape[1]
    bufs = (rows_a, rows_b, rows_c)
    gsems = (gsem_a, gsem_b, gsem_c)
    wsems = (wsem_a, wsem_b, wsem_c)

    def out_slice(j):
        return out_hbm.at[pl.ds(wid * n_per_w + j * K, K)]

    pltpu.sync_copy(x_hbm.at[wid], idx_v)
    pltpu.async_copy(pe_hbm.at[idx_v.at[0]], bufs[0], gsems[0])
    pltpu.async_copy(pe_hbm.at[idx_v.at[1]], bufs[1], gsems[1])

    n_main = n_chunks - 2
    assert n_main % NBUF == 0

    @pl.loop(0, n_main, step=NBUF)
    def _(j):
        for p in range(NBUF):
            jj = j + p
            q = (p + 2) % NBUF
            pltpu.make_async_copy(
                pe_hbm.at[idx_v.at[jj]], bufs[p], gsems[p]).wait()
            pltpu.async_copy(bufs[p], out_slice(jj), wsems[p])

            @pl.when(jj >= 1)
            def _():
                pltpu.make_async_copy(
                    bufs[q], out_slice(jj - 1), wsems[q]).wait()

            pltpu.async_copy(pe_hbm.at[idx_v.at[jj + 2]], bufs[q], gsems[q])

    for jj in (n_chunks - 2, n_chunks - 1):
        p = jj % NBUF
        pltpu.make_async_copy(
            pe_hbm.at[idx_v.at[jj]], bufs[p], gsems[p]).wait()
        pltpu.async_copy(bufs[p], out_slice(jj), wsems[p])

    for jj in (n_chunks - 3, n_chunks - 2, n_chunks - 1):
        p = jj % NBUF
        pltpu.make_async_copy(bufs[p], out_slice(jj), wsems[p]).wait()


def kernel(x, pe):
    b, l = x.shape
    total = b * l
    n_per_w = total // NW
    n_chunks = n_per_w // K
    x_resh = x.reshape(NW, n_chunks, K).astype(jnp.int32)

    mesh = plsc.VectorSubcoreMesh(core_axis_name="c", subcore_axis_name="s")
    out = pl.kernel(
        _gather_body,
        out_type=jax.ShapeDtypeStruct((total, D_MODEL), jnp.float32),
        mesh=mesh,
        scratch_types=[
            pltpu.VMEM((n_chunks, K), jnp.int32),
            pltpu.VMEM((K, D_MODEL), jnp.float32),
            pltpu.VMEM((K, D_MODEL), jnp.float32),
            pltpu.VMEM((K, D_MODEL), jnp.float32),
            pltpu.SemaphoreType.DMA,
            pltpu.SemaphoreType.DMA,
            pltpu.SemaphoreType.DMA,
            pltpu.SemaphoreType.DMA,
            pltpu.SemaphoreType.DMA,
            pltpu.SemaphoreType.DMA,
        ],
    )(x_resh, pe)
    return out.reshape(b, l, D_MODEL)
